# Initial kernel scaffold; baseline (speedup 1.0000x reference)
import jax
import jax.numpy as jnp
from jax.experimental import pallas as pl


def kernel(data):
    raise NotImplementedError("write your pallas kernel here")


# speedup vs baseline: 48.4444x; 48.4444x over previous
import functools

import jax
import jax.numpy as jnp
from jax import lax
from jax.experimental import pallas as pl
from jax.experimental.pallas import tpu as pltpu
from jax.experimental.pallas import tpu_sc as plsc

H = 512
W = 512
NPIX = H * W
NIMG = 96
NWORK = 32
PPW = NPIX // NWORK
CHUNK = 256
NCHUNK = PPW // CHUNK
LANES = 16
IMG_GROUPS = NIMG // LANES

_LOG_POLAR_DISTANCE = 2.0


def _sc_body(dataT, i0h, i1h, i2h, i3h, w0h, w1h, w2h, w3h, out,
             i0s, i1s, i2s, i3s, w0s, w1s, w2s, w3s,
             g0, g1, g2, g3, osb, sem):
    c = lax.axis_index("c")
    s = lax.axis_index("s")
    wid = s * 2 + c
    base = wid * PPW

    def chunk_body(ci, carry):
        start = pl.multiple_of(base + ci * CHUNK, CHUNK)
        sl = pl.ds(start, CHUNK)
        pltpu.sync_copy(i0h.at[sl], i0s)
        pltpu.sync_copy(i1h.at[sl], i1s)
        pltpu.sync_copy(i2h.at[sl], i2s)
        pltpu.sync_copy(i3h.at[sl], i3s)
        pltpu.sync_copy(w0h.at[sl], w0s)
        pltpu.sync_copy(w1h.at[sl], w1s)
        pltpu.sync_copy(w2h.at[sl], w2s)
        pltpu.sync_copy(w3h.at[sl], w3s)
        cp0 = pltpu.async_copy(dataT.at[i0s], g0, sem)
        cp1 = pltpu.async_copy(dataT.at[i1s], g1, sem)
        cp2 = pltpu.async_copy(dataT.at[i2s], g2, sem)
        cp3 = pltpu.async_copy(dataT.at[i3s], g3, sem)
        cp0.wait()
        cp1.wait()
        cp2.wait()
        cp3.wait()

        def pixgrp(pg, pcarry):
            pbase = pg * LANES
            wv0 = w0s[pl.ds(pbase, LANES)]
            wv1 = w1s[pl.ds(pbase, LANES)]
            wv2 = w2s[pl.ds(pbase, LANES)]
            wv3 = w3s[pl.ds(pbase, LANES)]
            for j in range(LANES):
                p = pbase + j
                a0 = jnp.full((LANES,), wv0[j], jnp.float32)
                a1 = jnp.full((LANES,), wv1[j], jnp.float32)
                a2 = jnp.full((LANES,), wv2[j], jnp.float32)
                a3 = jnp.full((LANES,), wv3[j], jnp.float32)
                for gb in range(IMG_GROUPS):
                    ls = pl.ds(gb * LANES, LANES)
                    osb[p, ls] = (a0 * g0[p, ls] + a1 * g1[p, ls]
                                  + a2 * g2[p, ls] + a3 * g3[p, ls])
            return pcarry

        lax.fori_loop(0, CHUNK // LANES, pixgrp, 0, unroll=False)
        pltpu.sync_copy(osb, out.at[sl])
        return carry

    lax.fori_loop(0, NCHUNK, chunk_body, 0, unroll=False)


@functools.partial(jax.jit, static_argnums=())
def _run(dataT, i0, i1, i2, i3, w0, w1, w2, w3):
    mesh = plsc.VectorSubcoreMesh(core_axis_name="c", subcore_axis_name="s")
    f = pl.kernel(
        _sc_body,
        mesh=mesh,
        compiler_params=pltpu.CompilerParams(use_tc_tiling_on_sc=False),
        out_type=jax.ShapeDtypeStruct((NPIX, NIMG), jnp.float32),
        scratch_types=[
            pltpu.VMEM((CHUNK,), jnp.int32),
            pltpu.VMEM((CHUNK,), jnp.int32),
            pltpu.VMEM((CHUNK,), jnp.int32),
            pltpu.VMEM((CHUNK,), jnp.int32),
            pltpu.VMEM((CHUNK,), jnp.float32),
            pltpu.VMEM((CHUNK,), jnp.float32),
            pltpu.VMEM((CHUNK,), jnp.float32),
            pltpu.VMEM((CHUNK,), jnp.float32),
            pltpu.VMEM((CHUNK, NIMG), jnp.float32),
            pltpu.VMEM((CHUNK, NIMG), jnp.float32),
            pltpu.VMEM((CHUNK, NIMG), jnp.float32),
            pltpu.VMEM((CHUNK, NIMG), jnp.float32),
            pltpu.VMEM((CHUNK, NIMG), jnp.float32),
            pltpu.SemaphoreType.DMA,
        ],
    )
    return f(dataT, i0, i1, i2, i3, w0, w1, w2, w3)


def _grid():
    max_r = jnp.log(
        jnp.linalg.norm(jnp.asarray((H, W), dtype=jnp.float32)) / 2.0
        * _LOG_POLAR_DISTANCE)
    theta, r = jnp.meshgrid(jnp.arange(H), jnp.arange(W), indexing="ij")
    theta = theta.astype(jnp.float32)
    r = r.astype(jnp.float32)
    X = jnp.exp(r * max_r / W) * jnp.cos(theta * 2.0 * jnp.pi / H)
    Y = jnp.exp(r * max_r / W) * jnp.sin(theta * 2.0 * jnp.pi / H)
    X = W / 2.0 + X
    Y = H / 2.0 - Y

    y_down = jnp.clip(Y.astype(jnp.int32), 0, H - 1)
    x_down = jnp.clip(X.astype(jnp.int32), 0, W - 1)
    y_up = jnp.clip(y_down + 1, 0, H - 1)
    x_up = jnp.clip(x_down + 1, 0, W - 1)

    dd = (Y - y_down) ** 2 + (X - x_down) ** 2
    du = (Y - y_down) ** 2 + (X - x_up) ** 2
    ud = (Y - y_up) ** 2 + (X - x_down) ** 2
    uu = (Y - y_up) ** 2 + (X - x_up) ** 2
    tot = dd + du + ud + uu

    i0 = (y_down * W + x_down).reshape(-1).astype(jnp.int32)
    i1 = (y_down * W + x_up).reshape(-1).astype(jnp.int32)
    i2 = (y_up * W + x_down).reshape(-1).astype(jnp.int32)
    i3 = (y_up * W + x_up).reshape(-1).astype(jnp.int32)
    w0 = (dd / tot).reshape(-1)
    w1 = (du / tot).reshape(-1)
    w2 = (ud / tot).reshape(-1)
    w3 = (uu / tot).reshape(-1)
    return i0, i1, i2, i3, w0, w1, w2, w3


def kernel(data):
    i0, i1, i2, i3, w0, w1, w2, w3 = _grid()
    dataT = data.reshape(NIMG, NPIX).transpose(1, 0)
    outT = _run(dataT, i0, i1, i2, i3, w0, w1, w2, w3)
    return outT.transpose(1, 0).reshape(data.shape)


# speedup vs baseline: 50.6537x; 1.0456x over previous
import functools

import jax
import jax.numpy as jnp
from jax import lax
from jax.experimental import pallas as pl
from jax.experimental.pallas import tpu as pltpu
from jax.experimental.pallas import tpu_sc as plsc

H = 512
W = 512
NPIX = H * W
NIMG = 96
NWORK = 32
PPW = NPIX // NWORK
CHUNK = 128
NCHUNK = PPW // CHUNK
LANES = 16
IMG_GROUPS = NIMG // LANES

_LOG_POLAR_DISTANCE = 2.0


def _sc_body(dataT, i0h, i1h, i2h, i3h, w0h, w1h, w2h, w3h, out,
             ia, ib, wa, wb, ga, gb, osb, sema, semb):
    c = lax.axis_index("c")
    s = lax.axis_index("s")
    wid = s * 2 + c
    base = wid * PPW

    idx_bufs = (ia, ib)
    w_bufs = (wa, wb)
    g_bufs = (ga, gb)
    sems = (sema, semb)

    def fire(buf, ci):
        start = pl.multiple_of(base + ci * CHUNK, CHUNK)
        sl = pl.ds(start, CHUNK)
        iv, wv, gv, sem = idx_bufs[buf], w_bufs[buf], g_bufs[buf], sems[buf]
        pltpu.sync_copy(i0h.at[sl], iv.at[0])
        pltpu.sync_copy(i1h.at[sl], iv.at[1])
        pltpu.sync_copy(i2h.at[sl], iv.at[2])
        pltpu.sync_copy(i3h.at[sl], iv.at[3])
        pltpu.sync_copy(w0h.at[sl], wv.at[0])
        pltpu.sync_copy(w1h.at[sl], wv.at[1])
        pltpu.sync_copy(w2h.at[sl], wv.at[2])
        pltpu.sync_copy(w3h.at[sl], wv.at[3])
        for k in range(4):
            pltpu.async_copy(dataT.at[iv.at[k]], gv.at[k], sem)

    def drain_blend_write(buf, ci):
        start = pl.multiple_of(base + ci * CHUNK, CHUNK)
        iv, wv, gv, sem = idx_bufs[buf], w_bufs[buf], g_bufs[buf], sems[buf]
        for k in range(4):
            pltpu.make_async_copy(dataT.at[iv.at[k]], gv.at[k], sem).wait()

        def pixgrp(pg, pcarry):
            pbase = pg * LANES
            psl = pl.ds(pbase, LANES)
            wv0 = wv[0, psl]
            wv1 = wv[1, psl]
            wv2 = wv[2, psl]
            wv3 = wv[3, psl]
            for j in range(LANES):
                p = pbase + j
                a0 = jnp.full((LANES,), wv0[j], jnp.float32)
                a1 = jnp.full((LANES,), wv1[j], jnp.float32)
                a2 = jnp.full((LANES,), wv2[j], jnp.float32)
                a3 = jnp.full((LANES,), wv3[j], jnp.float32)
                for g in range(IMG_GROUPS):
                    ls = pl.ds(g * LANES, LANES)
                    osb[p, ls] = (a0 * gv[0, p, ls] + a1 * gv[1, p, ls]
                                  + a2 * gv[2, p, ls] + a3 * gv[3, p, ls])
            return pcarry

        lax.fori_loop(0, CHUNK // LANES, pixgrp, 0, unroll=False)
        pltpu.sync_copy(osb, out.at[pl.ds(start, CHUNK)])

    fire(0, 0)

    def pair_body(i, carry):
        c0 = i * 2
        fire(1, c0 + 1)
        drain_blend_write(0, c0)

        @pl.when(c0 + 2 < NCHUNK)
        def _():
            fire(0, c0 + 2)

        drain_blend_write(1, c0 + 1)
        return carry

    lax.fori_loop(0, NCHUNK // 2, pair_body, 0, unroll=False)


@functools.partial(jax.jit, static_argnums=())
def _run(dataT, i0, i1, i2, i3, w0, w1, w2, w3):
    mesh = plsc.VectorSubcoreMesh(core_axis_name="c", subcore_axis_name="s")
    f = pl.kernel(
        _sc_body,
        mesh=mesh,
        compiler_params=pltpu.CompilerParams(use_tc_tiling_on_sc=False),
        out_type=jax.ShapeDtypeStruct((NPIX, NIMG), jnp.float32),
        scratch_types=[
            pltpu.VMEM((4, CHUNK), jnp.int32),
            pltpu.VMEM((4, CHUNK), jnp.int32),
            pltpu.VMEM((4, CHUNK), jnp.float32),
            pltpu.VMEM((4, CHUNK), jnp.float32),
            pltpu.VMEM((4, CHUNK, NIMG), jnp.float32),
            pltpu.VMEM((4, CHUNK, NIMG), jnp.float32),
            pltpu.VMEM((CHUNK, NIMG), jnp.float32),
            pltpu.SemaphoreType.DMA,
            pltpu.SemaphoreType.DMA,
        ],
    )
    return f(dataT, i0, i1, i2, i3, w0, w1, w2, w3)


def _grid():
    max_r = jnp.log(
        jnp.linalg.norm(jnp.asarray((H, W), dtype=jnp.float32)) / 2.0
        * _LOG_POLAR_DISTANCE)
    theta, r = jnp.meshgrid(jnp.arange(H), jnp.arange(W), indexing="ij")
    theta = theta.astype(jnp.float32)
    r = r.astype(jnp.float32)
    X = jnp.exp(r * max_r / W) * jnp.cos(theta * 2.0 * jnp.pi / H)
    Y = jnp.exp(r * max_r / W) * jnp.sin(theta * 2.0 * jnp.pi / H)
    X = W / 2.0 + X
    Y = H / 2.0 - Y

    y_down = jnp.clip(Y.astype(jnp.int32), 0, H - 1)
    x_down = jnp.clip(X.astype(jnp.int32), 0, W - 1)
    y_up = jnp.clip(y_down + 1, 0, H - 1)
    x_up = jnp.clip(x_down + 1, 0, W - 1)

    dd = (Y - y_down) ** 2 + (X - x_down) ** 2
    du = (Y - y_down) ** 2 + (X - x_up) ** 2
    ud = (Y - y_up) ** 2 + (X - x_down) ** 2
    uu = (Y - y_up) ** 2 + (X - x_up) ** 2
    tot = dd + du + ud + uu

    i0 = (y_down * W + x_down).reshape(-1).astype(jnp.int32)
    i1 = (y_down * W + x_up).reshape(-1).astype(jnp.int32)
    i2 = (y_up * W + x_down).reshape(-1).astype(jnp.int32)
    i3 = (y_up * W + x_up).reshape(-1).astype(jnp.int32)
    w0 = (dd / tot).reshape(-1)
    w1 = (du / tot).reshape(-1)
    w2 = (ud / tot).reshape(-1)
    w3 = (uu / tot).reshape(-1)
    return i0, i1, i2, i3, w0, w1, w2, w3


def kernel(data):
    i0, i1, i2, i3, w0, w1, w2, w3 = _grid()
    dataT = data.reshape(NIMG, NPIX).transpose(1, 0)
    outT = _run(dataT, i0, i1, i2, i3, w0, w1, w2, w3)
    return outT.transpose(1, 0).reshape(data.shape)


# speedup vs baseline: 51.2620x; 1.0120x over previous
import functools

import jax
import jax.numpy as jnp
from jax import lax
from jax.experimental import pallas as pl
from jax.experimental.pallas import tpu as pltpu
from jax.experimental.pallas import tpu_sc as plsc

H = 512
W = 512
NPIX = H * W
NIMG = 96
NWORK = 32
PPW = NPIX // NWORK
CHUNK = 128
NCHUNK = PPW // CHUNK
LANES = 16
IMG_GROUPS = NIMG // LANES
PK = 4 * CHUNK

_LOG_POLAR_DISTANCE = 2.0


def _blend(wv, gv, osb):
    def pixgrp(pg, pcarry):
        pbase = pg * LANES
        wv0 = wv[pl.ds(0 * CHUNK + pbase, LANES)]
        wv1 = wv[pl.ds(1 * CHUNK + pbase, LANES)]
        wv2 = wv[pl.ds(2 * CHUNK + pbase, LANES)]
        wv3 = wv[pl.ds(3 * CHUNK + pbase, LANES)]
        for j in range(LANES):
            p = pbase + j
            a0 = jnp.full((LANES,), wv0[j], jnp.float32)
            a1 = jnp.full((LANES,), wv1[j], jnp.float32)
            a2 = jnp.full((LANES,), wv2[j], jnp.float32)
            a3 = jnp.full((LANES,), wv3[j], jnp.float32)
            for g in range(IMG_GROUPS):
                ls = pl.ds(g * LANES, LANES)
                osb[p, ls] = (a0 * gv[0, p, ls] + a1 * gv[1, p, ls]
                              + a2 * gv[2, p, ls] + a3 * gv[3, p, ls])
        return pcarry

    lax.fori_loop(0, CHUNK // LANES, pixgrp, 0, unroll=False)


def _sc_body(packi, packw, dataT, out,
             pia, pib, pwa, pwb, ga, gb, osb,
             sema, semb, semsa, semsb):
    c = lax.axis_index("c")
    s = lax.axis_index("s")
    wid = s * 2 + c
    cbase = wid * NCHUNK

    pi_bufs = (pia, pib)
    pw_bufs = (pwa, pwb)
    g_bufs = (ga, gb)
    sems = (sema, semb)
    ssems = (semsa, semsb)

    def stage(buf, ci):
        g0 = pl.multiple_of((cbase + ci) * PK, PK)
        pltpu.async_copy(packi.at[pl.ds(g0, PK)], pi_bufs[buf], ssems[buf])
        pltpu.async_copy(packw.at[pl.ds(g0, PK)], pw_bufs[buf], ssems[buf])

    def stage_wait(buf, ci):
        g0 = pl.multiple_of((cbase + ci) * PK, PK)
        pltpu.make_async_copy(packi.at[pl.ds(g0, PK)], pi_bufs[buf],
                              ssems[buf]).wait()
        pltpu.make_async_copy(packw.at[pl.ds(g0, PK)], pw_bufs[buf],
                              ssems[buf]).wait()

    def fire(buf):
        iv, gv, sem = pi_bufs[buf], g_bufs[buf], sems[buf]
        for k in range(4):
            pltpu.async_copy(dataT.at[iv.at[pl.ds(k * CHUNK, CHUNK)]],
                             gv.at[k], sem)

    def drain(buf):
        iv, gv, sem = pi_bufs[buf], g_bufs[buf], sems[buf]
        for k in range(4):
            pltpu.make_async_copy(dataT.at[iv.at[pl.ds(k * CHUNK, CHUNK)]],
                                  gv.at[k], sem).wait()

    def write_out(ci):
        start = pl.multiple_of((cbase + ci) * CHUNK, CHUNK)
        pltpu.sync_copy(osb, out.at[pl.ds(start, CHUNK)])

    stage(0, 0)
    stage(1, 1)
    stage_wait(0, 0)
    fire(0)
    stage_wait(1, 1)
    fire(1)

    def pair_body(i, carry):
        c0 = i * 2
        more = c0 + 2 < NCHUNK

        drain(0)

        @pl.when(more)
        def _():
            stage(0, c0 + 2)

        _blend(pwa, ga, osb)
        write_out(c0)

        @pl.when(more)
        def _():
            stage_wait(0, c0 + 2)
            fire(0)

        drain(1)

        @pl.when(more)
        def _():
            stage(1, c0 + 3)

        _blend(pwb, gb, osb)
        write_out(c0 + 1)

        @pl.when(more)
        def _():
            stage_wait(1, c0 + 3)
            fire(1)

        return carry

    lax.fori_loop(0, NCHUNK // 2, pair_body, 0, unroll=False)


@functools.partial(jax.jit, static_argnums=())
def _run(packi, packw, dataT):
    mesh = plsc.VectorSubcoreMesh(core_axis_name="c", subcore_axis_name="s")
    f = pl.kernel(
        _sc_body,
        mesh=mesh,
        compiler_params=pltpu.CompilerParams(use_tc_tiling_on_sc=False),
        out_type=jax.ShapeDtypeStruct((NPIX, NIMG), jnp.float32),
        scratch_types=[
            pltpu.VMEM((PK,), jnp.int32),
            pltpu.VMEM((PK,), jnp.int32),
            pltpu.VMEM((PK,), jnp.float32),
            pltpu.VMEM((PK,), jnp.float32),
            pltpu.VMEM((4, CHUNK, NIMG), jnp.float32),
            pltpu.VMEM((4, CHUNK, NIMG), jnp.float32),
            pltpu.VMEM((CHUNK, NIMG), jnp.float32),
            pltpu.SemaphoreType.DMA,
            pltpu.SemaphoreType.DMA,
            pltpu.SemaphoreType.DMA,
            pltpu.SemaphoreType.DMA,
        ],
    )
    return f(packi, packw, dataT)


def _grid():
    max_r = jnp.log(
        jnp.linalg.norm(jnp.asarray((H, W), dtype=jnp.float32)) / 2.0
        * _LOG_POLAR_DISTANCE)
    theta, r = jnp.meshgrid(jnp.arange(H), jnp.arange(W), indexing="ij")
    theta = theta.astype(jnp.float32)
    r = r.astype(jnp.float32)
    X = jnp.exp(r * max_r / W) * jnp.cos(theta * 2.0 * jnp.pi / H)
    Y = jnp.exp(r * max_r / W) * jnp.sin(theta * 2.0 * jnp.pi / H)
    X = W / 2.0 + X
    Y = H / 2.0 - Y

    y_down = jnp.clip(Y.astype(jnp.int32), 0, H - 1)
    x_down = jnp.clip(X.astype(jnp.int32), 0, W - 1)
    y_up = jnp.clip(y_down + 1, 0, H - 1)
    x_up = jnp.clip(x_down + 1, 0, W - 1)

    dd = (Y - y_down) ** 2 + (X - x_down) ** 2
    du = (Y - y_down) ** 2 + (X - x_up) ** 2
    ud = (Y - y_up) ** 2 + (X - x_down) ** 2
    uu = (Y - y_up) ** 2 + (X - x_up) ** 2
    tot = dd + du + ud + uu

    idx = jnp.stack([
        (y_down * W + x_down).reshape(-1),
        (y_down * W + x_up).reshape(-1),
        (y_up * W + x_down).reshape(-1),
        (y_up * W + x_up).reshape(-1),
    ]).astype(jnp.int32)
    wts = jnp.stack([
        (dd / tot).reshape(-1),
        (du / tot).reshape(-1),
        (ud / tot).reshape(-1),
        (uu / tot).reshape(-1),
    ])
    packi = (idx.reshape(4, NPIX // CHUNK, CHUNK)
             .transpose(1, 0, 2).reshape(-1))
    packw = (wts.reshape(4, NPIX // CHUNK, CHUNK)
             .transpose(1, 0, 2).reshape(-1))
    return packi, packw


def kernel(data):
    packi, packw = _grid()
    dataT = data.reshape(NIMG, NPIX).transpose(1, 0)
    outT = _run(packi, packw, dataT)
    return outT.transpose(1, 0).reshape(data.shape)


# speedup vs baseline: 54.3381x; 1.0600x over previous
import functools

import jax
import jax.numpy as jnp
from jax import lax
from jax.experimental import pallas as pl
from jax.experimental.pallas import tpu as pltpu
from jax.experimental.pallas import tpu_sc as plsc

H = 512
W = 512
NPIX = H * W
NIMG = 96
NWORK = 32
PPW = NPIX // NWORK
CHUNK = 128
NCHUNK = PPW // CHUNK
LANES = 16
IMG_GROUPS = NIMG // LANES
PK = 4 * CHUNK

RC = 256
BOX0 = 224
BOXW = 64
NBOX = BOXW * BOXW

_LOG_POLAR_DISTANCE = 2.0


def _blend(wv, gv, osb):
    def pixgrp(pg, pcarry):
        pbase = pg * LANES
        wv0 = wv[pl.ds(0 * CHUNK + pbase, LANES)]
        wv1 = wv[pl.ds(1 * CHUNK + pbase, LANES)]
        wv2 = wv[pl.ds(2 * CHUNK + pbase, LANES)]
        wv3 = wv[pl.ds(3 * CHUNK + pbase, LANES)]
        for j in range(LANES):
            p = pbase + j
            a0 = jnp.full((LANES,), wv0[j], jnp.float32)
            a1 = jnp.full((LANES,), wv1[j], jnp.float32)
            a2 = jnp.full((LANES,), wv2[j], jnp.float32)
            a3 = jnp.full((LANES,), wv3[j], jnp.float32)
            for g in range(IMG_GROUPS):
                ls = pl.ds(g * LANES, LANES)
                osb[p, ls] = (a0 * gv[0, p, ls] + a1 * gv[1, p, ls]
                              + a2 * gv[2, p, ls] + a3 * gv[3, p, ls])
        return pcarry

    lax.fori_loop(0, CHUNK // LANES, pixgrp, 0, unroll=False)


def _sc_body(packi, packw, dataT, tabc, out,
             pia, pib, pwa, pwb, ga, gb, osb,
             sema, semb, semsa, semsb):
    c = lax.axis_index("c")
    s = lax.axis_index("s")
    wid = s * 2 + c
    cbase = wid * NCHUNK

    pi_bufs = (pia, pib)
    pw_bufs = (pwa, pwb)
    g_bufs = (ga, gb)
    sems = (sema, semb)
    ssems = (semsa, semsb)

    def stage_fire(buf, ci):
        gid = cbase + ci
        g0 = pl.multiple_of(gid * PK, PK)
        iv, wv, gv = pi_bufs[buf], pw_bufs[buf], g_bufs[buf]
        pltpu.async_copy(packi.at[pl.ds(g0, PK)], iv, ssems[buf])
        pltpu.async_copy(packw.at[pl.ds(g0, PK)], wv, ssems[buf])
        pltpu.make_async_copy(packi.at[pl.ds(g0, PK)], iv, ssems[buf]).wait()
        pltpu.make_async_copy(packw.at[pl.ds(g0, PK)], wv, ssems[buf]).wait()
        central = lax.rem(gid, 4) < 2

        @pl.when(central)
        def _():
            for k in range(4):
                pltpu.async_copy(tabc.at[iv.at[pl.ds(k * CHUNK, CHUNK)]],
                                 gv.at[k], sems[buf])

        @pl.when(jnp.logical_not(central))
        def _():
            for k in range(4):
                pltpu.async_copy(dataT.at[iv.at[pl.ds(k * CHUNK, CHUNK)]],
                                 gv.at[k], sems[buf])

    def drain(buf):
        iv, gv, sem = pi_bufs[buf], g_bufs[buf], sems[buf]
        for k in range(4):
            pltpu.make_async_copy(dataT.at[iv.at[pl.ds(k * CHUNK, CHUNK)]],
                                  gv.at[k], sem).wait()

    def write_out(ci):
        start = pl.multiple_of((cbase + ci) * CHUNK, CHUNK)
        pltpu.sync_copy(osb, out.at[pl.ds(start, CHUNK)])

    stage_fire(0, 0)
    stage_fire(1, 1)

    def pair_body(i, carry):
        c0 = i * 2
        more = c0 + 2 < NCHUNK

        drain(0)
        _blend(pwa, ga, osb)
        write_out(c0)

        @pl.when(more)
        def _():
            stage_fire(0, c0 + 2)

        drain(1)
        _blend(pwb, gb, osb)
        write_out(c0 + 1)

        @pl.when(more)
        def _():
            stage_fire(1, c0 + 3)

        return carry

    lax.fori_loop(0, NCHUNK // 2, pair_body, 0, unroll=False)


@functools.partial(jax.jit, static_argnums=())
def _run(packi, packw, dataT, tabc):
    mesh = plsc.VectorSubcoreMesh(core_axis_name="c", subcore_axis_name="s")
    f = pl.kernel(
        _sc_body,
        mesh=mesh,
        compiler_params=pltpu.CompilerParams(use_tc_tiling_on_sc=False),
        out_type=jax.ShapeDtypeStruct((NPIX, NIMG), jnp.float32),
        scratch_types=[
            pltpu.VMEM((PK,), jnp.int32),
            pltpu.VMEM((PK,), jnp.int32),
            pltpu.VMEM((PK,), jnp.float32),
            pltpu.VMEM((PK,), jnp.float32),
            pltpu.VMEM((4, CHUNK, NIMG), jnp.float32),
            pltpu.VMEM((4, CHUNK, NIMG), jnp.float32),
            pltpu.VMEM((CHUNK, NIMG), jnp.float32),
            pltpu.SemaphoreType.DMA,
            pltpu.SemaphoreType.DMA,
            pltpu.SemaphoreType.DMA,
            pltpu.SemaphoreType.DMA,
        ],
    )
    return f(packi, packw, dataT, tabc)


def _grid():
    max_r = jnp.log(
        jnp.linalg.norm(jnp.asarray((H, W), dtype=jnp.float32)) / 2.0
        * _LOG_POLAR_DISTANCE)
    theta, r = jnp.meshgrid(jnp.arange(H), jnp.arange(W), indexing="ij")
    theta = theta.astype(jnp.float32)
    r = r.astype(jnp.float32)
    X = jnp.exp(r * max_r / W) * jnp.cos(theta * 2.0 * jnp.pi / H)
    Y = jnp.exp(r * max_r / W) * jnp.sin(theta * 2.0 * jnp.pi / H)
    X = W / 2.0 + X
    Y = H / 2.0 - Y

    y_down = jnp.clip(Y.astype(jnp.int32), 0, H - 1)
    x_down = jnp.clip(X.astype(jnp.int32), 0, W - 1)
    y_up = jnp.clip(y_down + 1, 0, H - 1)
    x_up = jnp.clip(x_down + 1, 0, W - 1)

    dd = (Y - y_down) ** 2 + (X - x_down) ** 2
    du = (Y - y_down) ** 2 + (X - x_up) ** 2
    ud = (Y - y_up) ** 2 + (X - x_down) ** 2
    uu = (Y - y_up) ** 2 + (X - x_up) ** 2
    tot = dd + du + ud + uu

    central = (jnp.arange(W)[None, :] < RC)

    def pack_idx(yy, xx):
        full = yy * W + xx
        boxed = (yy - BOX0) * BOXW + (xx - BOX0)
        return jnp.where(central, boxed, full).reshape(-1)

    idx = jnp.stack([
        pack_idx(y_down, x_down),
        pack_idx(y_down, x_up),
        pack_idx(y_up, x_down),
        pack_idx(y_up, x_up),
    ]).astype(jnp.int32)
    wts = jnp.stack([
        (dd / tot).reshape(-1),
        (du / tot).reshape(-1),
        (ud / tot).reshape(-1),
        (uu / tot).reshape(-1),
    ])
    packi = (idx.reshape(4, NPIX // CHUNK, CHUNK)
             .transpose(1, 0, 2).reshape(-1))
    packw = (wts.reshape(4, NPIX // CHUNK, CHUNK)
             .transpose(1, 0, 2).reshape(-1))
    return packi, packw


def kernel(data):
    packi, packw = _grid()
    d3 = data.reshape(NIMG, H, W)
    dataT = d3.reshape(NIMG, NPIX).transpose(1, 0)
    tabc = (d3[:, BOX0:BOX0 + BOXW, BOX0:BOX0 + BOXW]
            .reshape(NIMG, NBOX).transpose(1, 0))
    outT = _run(packi, packw, dataT, tabc)
    return outT.transpose(1, 0).reshape(data.shape)
